# Initial kernel scaffold; baseline (speedup 1.0000x reference)
import jax
import jax.numpy as jnp
from jax.experimental import pallas as pl


def kernel(x, byte_table, ngram_3, ngram_4, ngram_5, ngram_6, ngram_7, ngram_8, W, b):
    raise NotImplementedError("write your pallas kernel here")


# speedup vs baseline: 13.3818x; 13.3818x over previous
import functools

import jax
import jax.numpy as jnp
from jax import lax
from jax.experimental import pallas as pl
from jax.experimental.pallas import tpu as pltpu
from jax.experimental.pallas import tpu_sc as plsc

B = 4
S = 2048
H = 128
TAB = 500000
NSLOT = 7

_CMOD = (1, 256, 65536, 277216, 467296, 127776, 210656, 427936)
_WRAP = 448384

_NC = 2
_NS = 16
_NW = _NC * _NS

_CHUNK = 64
_CPB = S // _CHUNK
_TPW = (B * _CPB) // _NW


def _sc_gather_body(x_hbm, byte_hbm, t3, t4, t5, t6, t7, t8, out_hbm,
                    xv, idxv, rowsv, sem):
    wid = lax.axis_index("s") * _NC + lax.axis_index("c")
    pltpu.sync_copy(x_hbm, xv.at[pl.ds(0, B * S)])
    xv[pl.ds(B * S, 16)] = jnp.zeros((16,), jnp.int32)

    tables = (t3, t4, t5, t6, t7, t8, byte_hbm)
    c500k = jnp.full((16,), 500000, jnp.int32)
    zeros16 = jnp.zeros((16,), jnp.int32)
    wrap16 = jnp.full((16,), _WRAP, jnp.int32)

    for slot in range(NSLOT):
        n = slot + 3

        def task_body(j, slot=slot, n=n):
            task = wid * _TPW + j
            b = task // _CPB
            t0 = (task - b * _CPB) * _CHUNK
            base = b * S + t0
            for g in range(_CHUNK // 16):
                off = base + g * 16
                if slot == 6:
                    h = xv[pl.ds(off, 16)]
                else:
                    acc = xv[pl.ds(off, 16)]
                    for i in range(1, n):
                        acc = acc + xv[pl.ds(off + i, 16)] * _CMOD[i]
                    if n == 8:
                        x7 = xv[pl.ds(off + 7, 16)]
                        acc = acc + jnp.where(x7 >= 128, wrap16, zeros16)
                    h = jnp.maximum(lax.rem(acc, c500k), zeros16)
                idxv[pl.ds(g * 16, 16)] = h
            pltpu.async_copy(tables[slot].at[idxv], rowsv, sem).wait()
            out_base = (slot * B + b) * S + t0
            pltpu.sync_copy(rowsv, out_hbm.at[pl.ds(out_base, _CHUNK)])

        for j in range(_TPW):
            task_body(j)


@functools.cache
def _build_sc_gather():
    mesh = plsc.VectorSubcoreMesh(core_axis_name="c", subcore_axis_name="s")
    return pl.kernel(
        _sc_gather_body,
        out_type=jax.ShapeDtypeStruct((NSLOT * B * S, H), jnp.float32),
        mesh=mesh,
        scratch_types=[
            pltpu.VMEM((B * S + 16,), jnp.int32),
            pltpu.VMEM((_CHUNK,), jnp.int32),
            pltpu.VMEM((_CHUNK, H), jnp.float32),
            pltpu.SemaphoreType.DMA,
        ],
    )


_TBLK = 512


def _tc_body(f_ref, w_ref, b_ref, o_ref):
    tb = pl.program_id(1)
    acc = f_ref[6, 0] + b_ref[0][None, :]
    row = lax.broadcasted_iota(jnp.int32, (_TBLK, H), 0) + tb * _TBLK
    for k in range(6):
        n = k + 3
        f = f_ref[k, 0]
        f = jnp.where(row <= S - n, f, 0.0)
        wk = w_ref[:, k * H:(k + 1) * H]
        acc = acc + lax.dot_general(
            f, wk, (((1,), (1,)), ((), ())),
            preferred_element_type=jnp.float32)
    o_ref[0] = acc


_tc_project = pl.pallas_call(
    _tc_body,
    grid=(B, S // _TBLK),
    in_specs=[
        pl.BlockSpec((NSLOT, 1, _TBLK, H),
                     lambda b, t: (jnp.int32(0), b, t, jnp.int32(0))),
        pl.BlockSpec((H, 6 * H), lambda b, t: (jnp.int32(0), jnp.int32(0))),
        pl.BlockSpec((1, H), lambda b, t: (jnp.int32(0), jnp.int32(0))),
    ],
    out_specs=pl.BlockSpec((1, _TBLK, H), lambda b, t: (b, t, jnp.int32(0))),
    out_shape=jax.ShapeDtypeStruct((B, S, H), jnp.float32),
)


def kernel(x, byte_table, ngram_3, ngram_4, ngram_5, ngram_6, ngram_7,
           ngram_8, W, b):
    x32 = x.astype(jnp.int32).reshape(B * S)
    feats = _build_sc_gather()(x32, byte_table, ngram_3, ngram_4, ngram_5,
                               ngram_6, ngram_7, ngram_8)
    feats = feats.reshape(NSLOT, B, S, H)
    return _tc_project(feats, W, b.reshape(1, H))


# speedup vs baseline: 16.1276x; 1.2052x over previous
import functools

import jax
import jax.numpy as jnp
from jax import lax
from jax.experimental import pallas as pl
from jax.experimental.pallas import tpu as pltpu
from jax.experimental.pallas import tpu_sc as plsc

B = 4
S = 2048
H = 128
TAB = 500000
NSLOT = 7

_CMOD = (1, 256, 65536, 277216, 467296, 127776, 210656, 427936)
_WRAP = 448384

_NC = 2
_NS = 16
_NW = _NC * _NS

_CHUNK = 64
_CPB = S // _CHUNK
_TPW = (B * _CPB) // _NW


def _sc_gather_body(x_hbm, byte_hbm, t3, t4, t5, t6, t7, t8, out_hbm,
                    xv, idxv0, idxv1, rowsv0, rowsv1,
                    gsem0, gsem1, wsem0, wsem1):
    wid = lax.axis_index("s") * _NC + lax.axis_index("c")
    pltpu.sync_copy(x_hbm, xv.at[pl.ds(0, B * S)])
    xv[pl.ds(B * S, 16)] = jnp.zeros((16,), jnp.int32)

    tables = (t3, t4, t5, t6, t7, t8, byte_hbm)
    c500k = jnp.full((16,), 500000, jnp.int32)
    zeros16 = jnp.zeros((16,), jnp.int32)
    wrap16 = jnp.full((16,), _WRAP, jnp.int32)

    idxv = (idxv0, idxv1)
    rowsv = (rowsv0, rowsv1)
    gsem = (gsem0, gsem1)
    wsem = (wsem0, wsem1)

    def compute_idx(slot, n, j, p):
        task = wid * _TPW + j
        b = task // _CPB
        t0 = (task - b * _CPB) * _CHUNK
        base = b * S + t0
        for g in range(_CHUNK // 16):
            off = base + g * 16
            if slot == 6:
                h = xv[pl.ds(off, 16)]
            else:
                acc = xv[pl.ds(off, 16)]
                for i in range(1, n):
                    acc = acc + xv[pl.ds(off + i, 16)] * _CMOD[i]
                if n == 8:
                    x7 = xv[pl.ds(off + 7, 16)]
                    acc = acc + jnp.where(x7 >= 128, wrap16, zeros16)
                h = jnp.maximum(lax.rem(acc, c500k), zeros16)
            idxv[p][pl.ds(g * 16, 16)] = h
        return (slot * B + b) * S + t0

    tasks = [(slot, j) for slot in range(NSLOT) for j in range(_TPW)]
    pend_g = [None, None]
    pend_w = [None, None]
    out_base_of = [None, None]
    for i, (slot, j) in enumerate(tasks):
        p = i % 2
        if pend_w[p] is not None:
            pend_w[p].wait()
            pend_w[p] = None
        out_base_of[p] = compute_idx(slot, slot + 3, j, p)
        pend_g[p] = pltpu.async_copy(tables[slot].at[idxv[p]], rowsv[p],
                                     gsem[p])
        q = 1 - p
        if pend_g[q] is not None:
            pend_g[q].wait()
            pend_g[q] = None
            pend_w[q] = pltpu.async_copy(
                rowsv[q], out_hbm.at[pl.ds(out_base_of[q], _CHUNK)], wsem[q])
    p = (len(tasks) - 1) % 2
    pend_g[p].wait()
    pend_w[p] = pltpu.async_copy(
        rowsv[p], out_hbm.at[pl.ds(out_base_of[p], _CHUNK)], wsem[p])
    for q in range(2):
        if pend_w[q] is not None:
            pend_w[q].wait()


@functools.cache
def _build_sc_gather():
    mesh = plsc.VectorSubcoreMesh(core_axis_name="c", subcore_axis_name="s")
    return pl.kernel(
        _sc_gather_body,
        out_type=jax.ShapeDtypeStruct((NSLOT * B * S, H), jnp.float32),
        mesh=mesh,
        scratch_types=[
            pltpu.VMEM((B * S + 16,), jnp.int32),
            pltpu.VMEM((_CHUNK,), jnp.int32),
            pltpu.VMEM((_CHUNK,), jnp.int32),
            pltpu.VMEM((_CHUNK, H), jnp.float32),
            pltpu.VMEM((_CHUNK, H), jnp.float32),
            pltpu.SemaphoreType.DMA,
            pltpu.SemaphoreType.DMA,
            pltpu.SemaphoreType.DMA,
            pltpu.SemaphoreType.DMA,
        ],
    )


_TBLK = 512


def _tc_body(f_ref, w_ref, b_ref, o_ref):
    tb = pl.program_id(1)
    acc = f_ref[6, 0] + b_ref[0][None, :]
    row = lax.broadcasted_iota(jnp.int32, (_TBLK, H), 0) + tb * _TBLK
    for k in range(6):
        n = k + 3
        f = f_ref[k, 0]
        f = jnp.where(row <= S - n, f, 0.0)
        wk = w_ref[:, k * H:(k + 1) * H]
        acc = acc + lax.dot_general(
            f, wk, (((1,), (1,)), ((), ())),
            preferred_element_type=jnp.float32)
    o_ref[0] = acc


_tc_project = pl.pallas_call(
    _tc_body,
    grid=(B, S // _TBLK),
    in_specs=[
        pl.BlockSpec((NSLOT, 1, _TBLK, H),
                     lambda b, t: (jnp.int32(0), b, t, jnp.int32(0))),
        pl.BlockSpec((H, 6 * H), lambda b, t: (jnp.int32(0), jnp.int32(0))),
        pl.BlockSpec((1, H), lambda b, t: (jnp.int32(0), jnp.int32(0))),
    ],
    out_specs=pl.BlockSpec((1, _TBLK, H), lambda b, t: (b, t, jnp.int32(0))),
    out_shape=jax.ShapeDtypeStruct((B, S, H), jnp.float32),
)


def kernel(x, byte_table, ngram_3, ngram_4, ngram_5, ngram_6, ngram_7,
           ngram_8, W, b):
    x32 = x.astype(jnp.int32).reshape(B * S)
    feats = _build_sc_gather()(x32, byte_table, ngram_3, ngram_4, ngram_5,
                               ngram_6, ngram_7, ngram_8)
    feats = feats.reshape(NSLOT, B, S, H)
    return _tc_project(feats, W, b.reshape(1, H))


# speedup vs baseline: 17.1869x; 1.0657x over previous
import functools

import jax
import jax.numpy as jnp
from jax import lax
from jax.experimental import pallas as pl
from jax.experimental.pallas import tpu as pltpu
from jax.experimental.pallas import tpu_sc as plsc

B = 4
S = 2048
H = 128
TAB = 500000
NSLOT = 7

_CMOD = (1, 256, 65536, 277216, 467296, 127776, 210656, 427936)
_WRAP = 448384

_NC = 2
_NS = 16
_NW = _NC * _NS

_CHUNK = 128
_CPB = S // _CHUNK
_TPW = (B * _CPB) // _NW
_NBUF = 3


def _sc_gather_body(x_hbm, byte_hbm, t3, t4, t5, t6, t7, t8, out_hbm,
                    xv, idxv0, idxv1, idxv2, rowsv0, rowsv1, rowsv2,
                    gsem0, gsem1, gsem2, wsem0, wsem1, wsem2):
    wid = lax.axis_index("s") * _NC + lax.axis_index("c")
    pltpu.sync_copy(x_hbm, xv.at[pl.ds(0, B * S)])
    xv[pl.ds(B * S, 16)] = jnp.zeros((16,), jnp.int32)

    tables = (t3, t4, t5, t6, t7, t8, byte_hbm)
    c500k = jnp.full((16,), 500000, jnp.int32)
    zeros16 = jnp.zeros((16,), jnp.int32)
    wrap16 = jnp.full((16,), _WRAP, jnp.int32)

    idxv = (idxv0, idxv1, idxv2)
    rowsv = (rowsv0, rowsv1, rowsv2)
    gsem = (gsem0, gsem1, gsem2)
    wsem = (wsem0, wsem1, wsem2)

    def compute_idx(slot, n, j, p):
        task = wid * _TPW + j
        b = task // _CPB
        t0 = (task - b * _CPB) * _CHUNK
        base = b * S + t0
        for g in range(_CHUNK // 16):
            off = base + g * 16
            if slot == 6:
                h = xv[pl.ds(off, 16)]
            else:
                acc = xv[pl.ds(off, 16)]
                for i in range(1, n):
                    acc = acc + xv[pl.ds(off + i, 16)] * _CMOD[i]
                if n == 8:
                    x7 = xv[pl.ds(off + 7, 16)]
                    acc = acc + jnp.where(x7 >= 128, wrap16, zeros16)
                h = jnp.maximum(lax.rem(acc, c500k), zeros16)
            idxv[p][pl.ds(g * 16, 16)] = h
        return (slot * B + b) * S + t0

    tasks = [(slot, j) for slot in range(NSLOT) for j in range(_TPW)]
    pend_g = [None] * _NBUF
    pend_w = [None] * _NBUF
    out_base_of = [None] * _NBUF
    for i, (slot, j) in enumerate(tasks):
        p = i % _NBUF
        if pend_w[p] is not None:
            pend_w[p].wait()
            pend_w[p] = None
        out_base_of[p] = compute_idx(slot, slot + 3, j, p)
        pend_g[p] = pltpu.async_copy(tables[slot].at[idxv[p]], rowsv[p],
                                     gsem[p])
        q = (i - 1) % _NBUF
        if i >= 1 and pend_g[q] is not None:
            pend_g[q].wait()
            pend_g[q] = None
            pend_w[q] = pltpu.async_copy(
                rowsv[q], out_hbm.at[pl.ds(out_base_of[q], _CHUNK)], wsem[q])
    p = (len(tasks) - 1) % _NBUF
    pend_g[p].wait()
    pend_w[p] = pltpu.async_copy(
        rowsv[p], out_hbm.at[pl.ds(out_base_of[p], _CHUNK)], wsem[p])
    for q in range(_NBUF):
        if pend_w[q] is not None:
            pend_w[q].wait()


@functools.cache
def _build_sc_gather():
    mesh = plsc.VectorSubcoreMesh(core_axis_name="c", subcore_axis_name="s")
    return pl.kernel(
        _sc_gather_body,
        out_type=jax.ShapeDtypeStruct((NSLOT * B * S, H), jnp.float32),
        mesh=mesh,
        scratch_types=(
            [pltpu.VMEM((B * S + 16,), jnp.int32)]
            + [pltpu.VMEM((_CHUNK,), jnp.int32) for _ in range(_NBUF)]
            + [pltpu.VMEM((_CHUNK, H), jnp.float32) for _ in range(_NBUF)]
            + [pltpu.SemaphoreType.DMA for _ in range(2 * _NBUF)]
        ),
    )


_TBLK = 512


def _tc_body(f_ref, w_ref, b_ref, o_ref):
    tb = pl.program_id(1)
    acc = f_ref[6, 0] + b_ref[0][None, :]
    row = lax.broadcasted_iota(jnp.int32, (_TBLK, H), 0) + tb * _TBLK
    for k in range(6):
        n = k + 3
        f = f_ref[k, 0]
        f = jnp.where(row <= S - n, f, 0.0)
        wk = w_ref[:, k * H:(k + 1) * H]
        acc = acc + lax.dot_general(
            f, wk, (((1,), (1,)), ((), ())),
            preferred_element_type=jnp.float32)
    o_ref[0] = acc


_tc_project = pl.pallas_call(
    _tc_body,
    grid=(B, S // _TBLK),
    in_specs=[
        pl.BlockSpec((NSLOT, 1, _TBLK, H),
                     lambda b, t: (jnp.int32(0), b, t, jnp.int32(0))),
        pl.BlockSpec((H, 6 * H), lambda b, t: (jnp.int32(0), jnp.int32(0))),
        pl.BlockSpec((1, H), lambda b, t: (jnp.int32(0), jnp.int32(0))),
    ],
    out_specs=pl.BlockSpec((1, _TBLK, H), lambda b, t: (b, t, jnp.int32(0))),
    out_shape=jax.ShapeDtypeStruct((B, S, H), jnp.float32),
)


def kernel(x, byte_table, ngram_3, ngram_4, ngram_5, ngram_6, ngram_7,
           ngram_8, W, b):
    x32 = x.astype(jnp.int32).reshape(B * S)
    feats = _build_sc_gather()(x32, byte_table, ngram_3, ngram_4, ngram_5,
                               ngram_6, ngram_7, ngram_8)
    feats = feats.reshape(NSLOT, B, S, H)
    return _tc_project(feats, W, b.reshape(1, H))
